# baseline (speedup 1.0000x reference)
import functools
import jax
import jax.numpy as jnp
from jax import lax
from jax.experimental import pallas as pl
from jax.experimental.pallas import tpu as pltpu
from jax.experimental.pallas import tpu_sc as plsc

B = 16384
H = 20
D = 16
L = 16
NC = 2
NS = 16
NW = NC * NS
PER_W = B // NW
S = 128
NCHUNK = PER_W // S
ST = S * H
SD = S * D


def _score_body(user, item, it_in, uF, iF, u_bias, i_bias, t_embed, out,
                uidx, iidx, ueidx, ieidx, tidx, u_rows, i_rows, t_rows,
                ub, ib, prod_t, out_v, sem):
    wid = lax.axis_index("s") * NC + lax.axis_index("c")
    lanes = lax.iota(jnp.int32, L)

    for j in range(NCHUNK):
        base = wid * PER_W + j * S
        pltpu.sync_copy(user.at[pl.ds(base, S)], uidx)
        pltpu.sync_copy(item.at[pl.ds(base, S)], iidx)
        pltpu.sync_copy(it_in.at[pl.ds(base * H, ST)], tidx)

        def eidx(e, carry):
            ecol = jnp.full((L,), e, jnp.int32)
            ueidx[pl.ds(e * D, D)] = plsc.load_gather(uidx, [ecol]) * D + lanes
            ieidx[pl.ds(e * D, D)] = plsc.load_gather(iidx, [ecol]) * D + lanes
            return carry

        lax.fori_loop(0, S, eidx, 0)

        cps = [
            pltpu.async_copy(t_embed.at[tidx], t_rows, sem),
            pltpu.async_copy(uF.at[ueidx], u_rows, sem),
            pltpu.async_copy(iF.at[ieidx], i_rows, sem),
            pltpu.async_copy(u_bias.at[uidx], ub, sem),
            pltpu.async_copy(i_bias.at[iidx], ib, sem),
        ]
        for cp in cps:
            cp.wait()

        def example(e, carry):
            tb = e * H
            acc = t_rows[tb, :]
            for h in range(1, H):
                acc = acc + t_rows[tb + h, :]
            itv = i_rows[pl.ds(e * D, D)] + acc * (1.0 / H)
            prod_t[pl.ds(e * D, D)] = u_rows[pl.ds(e * D, D)] * itv
            return carry

        lax.fori_loop(0, S, example, 0)

        def group(g, carry):
            acc = plsc.load_gather(prod_t, [lanes * D + g * (L * D)])
            for d in range(1, D):
                acc = acc + plsc.load_gather(prod_t, [lanes * D + (g * (L * D) + d)])
            out_v[pl.ds(g * L, L)] = acc + ub[pl.ds(g * L, L)] + ib[pl.ds(g * L, L)]
            return carry

        lax.fori_loop(0, S // L, group, 0)
        pltpu.sync_copy(out_v, out.at[pl.ds(base, S)])


@functools.lru_cache(maxsize=1)
def _score_call():
  return pl.kernel(
    _score_body,
    out_type=jax.ShapeDtypeStruct((B,), jnp.float32),
    mesh=plsc.VectorSubcoreMesh(core_axis_name="c", subcore_axis_name="s",
                                num_cores=NC, num_subcores=NS),
    scratch_types=[
        pltpu.VMEM((S,), jnp.int32),
        pltpu.VMEM((S,), jnp.int32),
        pltpu.VMEM((SD,), jnp.int32),
        pltpu.VMEM((SD,), jnp.int32),
        pltpu.VMEM((ST,), jnp.int32),
        pltpu.VMEM((SD,), jnp.float32),
        pltpu.VMEM((SD,), jnp.float32),
        pltpu.VMEM((ST, D), jnp.float32),
        pltpu.VMEM((S,), jnp.float32),
        pltpu.VMEM((S,), jnp.float32),
        pltpu.VMEM((SD,), jnp.float32),
        pltpu.VMEM((S,), jnp.float32),
        pltpu.SemaphoreType.DMA,
    ],
    compiler_params=pltpu.CompilerParams(needs_layout_passes=False,
                                         use_tc_tiling_on_sc=False),
  )


def _flatten(t):
    return t.reshape(-1) + 0.0


@jax.jit
def kernel(user, item, it_in, it_off, u_bias, i_bias, u_embed, i_embed, t_embed):
    del it_off
    return _score_call()(user, item, it_in,
                         _flatten(u_embed), _flatten(i_embed),
                         u_bias.reshape(-1), i_bias.reshape(-1), t_embed)
